# baseline (speedup 1.0000x reference)
import jax
import jax.numpy as jnp
from jax.experimental import pallas as pl
from jax.experimental.pallas import tpu as pltpu

B, S, D = 4, 2048, 1024
ROWS = B * S
CR = 512
NCH = ROWS // CR
NSLOT = 12


def _body(x_hbm, pe_hbm, o_hbm, pebuf, xb, ob, sempe, semx, semo):
    dpe = pltpu.async_copy(pe_hbm, pebuf, sempe)

    def start_in(c):
        slot = c % NSLOT
        return pltpu.async_copy(
            x_hbm.at[pl.ds(c * CR, CR), :], xb.at[slot], semx.at[slot])

    in_descs = {c: start_in(c) for c in range(NSLOT)}
    out_descs = {}
    dpe.wait()
    for c in range(NCH):
        slot = c % NSLOT
        in_descs[c].wait()
        if c >= NSLOT:
            out_descs[c - NSLOT].wait()
        ps = (c * CR) % S
        ob[slot] = xb[slot] + pebuf[ps:ps + CR, :]
        out_descs[c] = pltpu.async_copy(
            ob.at[slot], o_hbm.at[pl.ds(c * CR, CR), :], semo.at[slot])
        if c + NSLOT < NCH:
            in_descs[c + NSLOT] = start_in(c + NSLOT)
    for c in range(NCH - NSLOT, NCH):
        out_descs[c].wait()


def _tc_add(x, pos_embedding):
    return pl.pallas_call(
        _body,
        grid=(),
        in_specs=[
            pl.BlockSpec(memory_space=pl.ANY),
            pl.BlockSpec(memory_space=pl.ANY),
        ],
        out_specs=pl.BlockSpec(memory_space=pl.ANY),
        out_shape=jax.ShapeDtypeStruct((ROWS, D), jnp.float32),
        scratch_shapes=[
            pltpu.VMEM((S, D), jnp.float32),
            pltpu.VMEM((NSLOT, CR, D), jnp.float32),
            pltpu.VMEM((NSLOT, CR, D), jnp.float32),
            pltpu.SemaphoreType.DMA,
            pltpu.SemaphoreType.DMA((NSLOT,)),
            pltpu.SemaphoreType.DMA((NSLOT,)),
        ],
    )(x.reshape(ROWS, D), pos_embedding)


def kernel(x, pos_embedding):
    return _tc_add(x, pos_embedding).reshape(x.shape)


# speedup vs baseline: 1.0399x; 1.0399x over previous
import jax
import jax.numpy as jnp
from jax.experimental import pallas as pl
from jax.experimental.pallas import tpu as pltpu

B, S, D = 4, 2048, 1024
ROWS = B * S
CR = 1024
NCH = ROWS // CR
NSLOT = 6
HS = S // 2


def _body(x_hbm, pe_hbm, o_hbm, pebuf, xb, ob, sempe, semx, semo):
    pe_descs = [
        pltpu.async_copy(pe_hbm.at[pl.ds(h * HS, HS), :],
                         pebuf.at[pl.ds(h * HS, HS), :], sempe.at[h])
        for h in range(2)
    ]

    def start_in(c):
        slot = c % NSLOT
        return pltpu.async_copy(
            x_hbm.at[pl.ds(c * CR, CR), :], xb.at[slot], semx.at[slot])

    in_descs = {c: start_in(c) for c in range(min(NSLOT, NCH))}
    out_descs = {}
    for c in range(NCH):
        slot = c % NSLOT
        in_descs[c].wait()
        if pe_descs:
            pe_descs.pop(0).wait()
        if c >= NSLOT:
            out_descs[c - NSLOT].wait()
        ps = (c * CR) % S
        ob[slot] = xb[slot] + pebuf[ps:ps + CR, :]
        out_descs[c] = pltpu.async_copy(
            ob.at[slot], o_hbm.at[pl.ds(c * CR, CR), :], semo.at[slot])
        if c + NSLOT < NCH:
            in_descs[c + NSLOT] = start_in(c + NSLOT)
    for c in range(max(0, NCH - NSLOT), NCH):
        out_descs[c].wait()


def _tc_add(x, pos_embedding):
    return pl.pallas_call(
        _body,
        grid=(),
        in_specs=[
            pl.BlockSpec(memory_space=pl.ANY),
            pl.BlockSpec(memory_space=pl.ANY),
        ],
        out_specs=pl.BlockSpec(memory_space=pl.ANY),
        out_shape=jax.ShapeDtypeStruct((ROWS, D), jnp.float32),
        scratch_shapes=[
            pltpu.VMEM((S, D), jnp.float32),
            pltpu.VMEM((NSLOT, CR, D), jnp.float32),
            pltpu.VMEM((NSLOT, CR, D), jnp.float32),
            pltpu.SemaphoreType.DMA((2,)),
            pltpu.SemaphoreType.DMA((NSLOT,)),
            pltpu.SemaphoreType.DMA((NSLOT,)),
        ],
    )(x.reshape(ROWS, D), pos_embedding)


def kernel(x, pos_embedding):
    return _tc_add(x, pos_embedding).reshape(x.shape)


# speedup vs baseline: 1.0432x; 1.0031x over previous
import jax
import jax.numpy as jnp
from jax.experimental import pallas as pl
from jax.experimental.pallas import tpu as pltpu

B, S, D = 4, 2048, 1024
ROWS = B * S
CR = 1024
NCH = ROWS // CR
NSLOT = 6
HS = S // 2


def _body(x_hbm, pe_hbm, o_hbm, pebuf, xb, ob, sempe, semx, semo):
    def start_pe(h):
        return pltpu.async_copy(pe_hbm.at[pl.ds(h * HS, HS), :],
                                pebuf.at[pl.ds(h * HS, HS), :], sempe.at[h])

    def start_in(c):
        slot = c % NSLOT
        return pltpu.async_copy(
            x_hbm.at[pl.ds(c * CR, CR), :], xb.at[slot], semx.at[slot])

    pe_descs = [start_pe(0)]
    in_descs = {0: start_in(0)}
    pe_descs.append(start_pe(1))
    in_descs.update({c: start_in(c) for c in range(1, min(NSLOT, NCH))})
    out_descs = {}
    for c in range(NCH):
        slot = c % NSLOT
        in_descs[c].wait()
        if pe_descs:
            pe_descs.pop(0).wait()
        if c >= NSLOT:
            out_descs[c - NSLOT].wait()
        ps = (c * CR) % S
        ob[slot] = xb[slot] + pebuf[ps:ps + CR, :]
        out_descs[c] = pltpu.async_copy(
            ob.at[slot], o_hbm.at[pl.ds(c * CR, CR), :], semo.at[slot])
        if c + NSLOT < NCH:
            in_descs[c + NSLOT] = start_in(c + NSLOT)
    for c in range(max(0, NCH - NSLOT), NCH):
        out_descs[c].wait()


def _tc_add(x, pos_embedding):
    return pl.pallas_call(
        _body,
        grid=(),
        in_specs=[
            pl.BlockSpec(memory_space=pl.ANY),
            pl.BlockSpec(memory_space=pl.ANY),
        ],
        out_specs=pl.BlockSpec(memory_space=pl.ANY),
        out_shape=jax.ShapeDtypeStruct((ROWS, D), jnp.float32),
        scratch_shapes=[
            pltpu.VMEM((S, D), jnp.float32),
            pltpu.VMEM((NSLOT, CR, D), jnp.float32),
            pltpu.VMEM((NSLOT, CR, D), jnp.float32),
            pltpu.SemaphoreType.DMA((2,)),
            pltpu.SemaphoreType.DMA((NSLOT,)),
            pltpu.SemaphoreType.DMA((NSLOT,)),
        ],
    )(x.reshape(ROWS, D), pos_embedding)


def kernel(x, pos_embedding):
    return _tc_add(x, pos_embedding).reshape(x.shape)
